# baseline (speedup 1.0000x reference)
import jax
import jax.numpy as jnp
from jax import lax
from jax.experimental import pallas as pl
from jax.experimental.pallas import tpu as pltpu
from jax.experimental.pallas import tpu_sc as plsc

_B, _C, _H, _W = 24, 112, 32, 88
_NPIX = _B * _H * _W
_NROW = _B * _H

_NC, _NS, _L = 2, 16, 16
_NW = _NC * _NS
_SC_PIX = 45056
_PIX_W = _SC_PIX // _NW
_ROWS_W = _PIX_W // _W
_CHUNK_ROWS = 4
_CHUNK = _CHUNK_ROWS * _W
_NCHUNK = _PIX_W // _CHUNK
_G0 = (0, 16, 32, 48, 64, 72)

_TC_PIX = _NPIX - _SC_PIX
_TC_GRID = 4
_TC_ROWS = (_NROW - _SC_PIX // _W) // _TC_GRID
_TC_BLK = _TC_ROWS * _W
_SC_ROWBLK = _SC_PIX // _W // _TC_ROWS

_LN2 = 0.6931471805599453
_SQRT2 = 1.4142135623730951


def _log_f32(x):
    bits = lax.bitcast_convert_type(x, jnp.int32)
    e = (bits >> 23) - 127
    m = lax.bitcast_convert_type(
        (bits & 0x007FFFFF) | 0x3F800000, jnp.float32)
    big = m > _SQRT2
    m = jnp.where(big, m * 0.5, m)
    e_f = e.astype(jnp.float32) + jnp.where(big, 1.0, 0.0)
    t = (m - 1.0) / (m + 1.0)
    t2 = t * t
    p = 2.0 + t2 * (2.0 / 3.0 + t2 * (0.4 + t2 * (2.0 / 7.0 + t2 * (2.0 / 9.0))))
    return e_f * _LN2 + t * p


def _sc_body(pred_hbm, gt_hbm, vm_hbm, out_hbm,
             gt_v, vm_v, blk_a, blk_b, stage_v, sem_a, sem_b):
    wid = lax.axis_index("s") * _NC + lax.axis_index("c")
    rowbase = wid * _ROWS_W
    pixbase = wid * _PIX_W

    pltpu.sync_copy(gt_hbm.at[pl.ds(rowbase, _ROWS_W)], gt_v)
    pltpu.sync_copy(vm_hbm.at[pl.ds(rowbase, _ROWS_W)], vm_v)

    lane = lax.iota(jnp.int32, _L)
    bufs = (blk_a, blk_b)
    sems = (sem_a, sem_b)

    def _copy(j, b):
        return pltpu.make_async_copy(
            pred_hbm.at[pl.ds(pixbase + j * _CHUNK, _CHUNK)],
            bufs[b], sems[b])

    _copy(0, 0).start()
    _copy(1, 1).start()

    zero = jnp.zeros((_L,), jnp.float32)

    def _pair(p, carry):
        ls, vs = carry
        for b in range(2):
            j = 2 * p + b
            _copy(j, b).wait()

            def _row(ri, carry):
                ls, vs = carry
                g_row = j * _CHUNK_ROWS + ri
                for gi, w0 in enumerate(_G0):
                    p_loc = ri * _W + w0 + lane
                    g = gt_v[g_row, pl.ds(w0, _L)]
                    bin_ = jnp.minimum(jnp.maximum(2 * g - 2, 0), _C - 1)
                    x = plsc.load_gather(bufs[b], [p_loc, bin_]) + 1e-8
                    vm = vm_v[g_row, pl.ds(w0, _L)]
                    if gi == len(_G0) - 1:
                        vm = jnp.where(lane >= 8, vm, 0.0)
                    ls = ls + vm * _log_f32(x)
                    vs = vs + vm
                return ls, vs

            ls, vs = lax.fori_loop(0, _CHUNK_ROWS, _row, (ls, vs))

            @pl.when(j + 2 < _NCHUNK)
            def _():
                _copy(j + 2, b).start()
        return ls, vs

    ls, vs = lax.fori_loop(0, _NCHUNK // 2, _pair, (zero, zero))

    stage_v[pl.ds(0, _L)] = ls
    pltpu.sync_copy(stage_v, out_hbm.at[0, wid])
    stage_v[pl.ds(0, _L)] = vs
    pltpu.sync_copy(stage_v, out_hbm.at[1, wid])


_sc_call = pl.kernel(
    _sc_body,
    out_type=jax.ShapeDtypeStruct((2, _NW, _L), jnp.float32),
    mesh=plsc.VectorSubcoreMesh(
        core_axis_name="c", subcore_axis_name="s",
        num_cores=_NC, num_subcores=_NS),
    compiler_params=pltpu.CompilerParams(needs_layout_passes=False),
    scratch_types=[
        pltpu.VMEM((_ROWS_W, _W), jnp.int32),
        pltpu.VMEM((_ROWS_W, _W), jnp.float32),
        pltpu.VMEM((_CHUNK, _C), jnp.float32),
        pltpu.VMEM((_CHUNK, _C), jnp.float32),
        pltpu.VMEM((_L,), jnp.float32),
        pltpu.SemaphoreType.DMA,
        pltpu.SemaphoreType.DMA,
    ],
)


def _tc_body(pred_ref, gt_ref, vm_ref, out_ref):
    p3 = pred_ref[...].reshape(_TC_ROWS, _W, _C)
    g = gt_ref[...]
    vm = vm_ref[...]
    bins = jnp.minimum(jnp.maximum(2 * g - 2, 0), _C - 1)
    v = jnp.take_along_axis(p3, bins[:, :, None], axis=2)[:, :, 0] + 1e-8
    ls = jnp.sum(vm * jnp.log(v))
    vs = jnp.sum(vm)
    lanes = lax.broadcasted_iota(jnp.int32, (1, 128), 1)
    row = jnp.where(lanes == 0, ls, jnp.where(lanes == 1, vs, 0.0))
    out_ref[pl.ds(pl.program_id(0), 1), :] = row


_tc_call = pl.pallas_call(
    _tc_body,
    grid=(_TC_GRID,),
    in_specs=[
        pl.BlockSpec((_TC_BLK, _C), lambda i: (_SC_PIX // _TC_BLK + i, 0)),
        pl.BlockSpec((_TC_ROWS, _W), lambda i: (_SC_ROWBLK + i, 0)),
        pl.BlockSpec((_TC_ROWS, _W), lambda i: (_SC_ROWBLK + i, 0)),
    ],
    out_specs=pl.BlockSpec((_TC_GRID, 128), lambda i: (0, 0)),
    out_shape=jax.ShapeDtypeStruct((_TC_GRID, 128), jnp.float32),
)


@jax.jit
def kernel(pred_depth, gt_depth_map, valid_mask):
    table = pred_depth.transpose(0, 2, 3, 1).reshape(_NPIX, _C)
    gt2 = gt_depth_map.reshape(_NROW, _W).astype(jnp.int32)
    vm2 = valid_mask.reshape(_NROW, _W)
    sc_parts = _sc_call(table, gt2, vm2)
    tc_parts = _tc_call(table, gt2, vm2)
    neg_wsum = jnp.sum(sc_parts[0]) + jnp.sum(tc_parts[:, 0])
    vm_sum = jnp.sum(sc_parts[1]) + jnp.sum(tc_parts[:, 1])
    weighted = -neg_wsum / jnp.maximum(vm_sum, 1e-12)
    return jnp.where(vm_sum > 0, weighted, jnp.float32(0.0))


# speedup vs baseline: 1.0494x; 1.0494x over previous
import jax
import jax.numpy as jnp
from jax import lax
from jax.experimental import pallas as pl
from jax.experimental.pallas import tpu as pltpu
from jax.experimental.pallas import tpu_sc as plsc

_B, _C, _H, _W = 24, 112, 32, 88
_NPIX = _B * _H * _W
_NROW = _B * _H

_NC, _NS, _L = 2, 16, 16
_NW = _NC * _NS
_SC_PIX = 45056
_PIX_W = _SC_PIX // _NW
_ROWS_W = _PIX_W // _W
_CHUNK_ROWS = 2
_CHUNK = _CHUNK_ROWS * _W
_NCHUNK = _PIX_W // _CHUNK
_RING = 4
_G0 = (0, 16, 32, 48, 64, 72)

_TC_PIX = _NPIX - _SC_PIX
_TC_GRID = 4
_TC_ROWS = (_NROW - _SC_PIX // _W) // _TC_GRID
_TC_BLK = _TC_ROWS * _W
_SC_ROWBLK = _SC_PIX // _W // _TC_ROWS

_LN2 = 0.6931471805599453
_SQRT2 = 1.4142135623730951


def _log_f32(x):
    bits = lax.bitcast_convert_type(x, jnp.int32)
    e = (bits >> 23) - 127
    m = lax.bitcast_convert_type(
        (bits & 0x007FFFFF) | 0x3F800000, jnp.float32)
    big = m > _SQRT2
    m = jnp.where(big, m * 0.5, m)
    e_f = e.astype(jnp.float32) + jnp.where(big, 1.0, 0.0)
    t = (m - 1.0) / (m + 1.0)
    t2 = t * t
    p = 2.0 + t2 * (2.0 / 3.0 + t2 * (0.4 + t2 * (2.0 / 7.0 + t2 * (2.0 / 9.0))))
    return e_f * _LN2 + t * p


def _sc_body(pred_hbm, gt_hbm, vm_hbm, out_hbm,
             gt_v, vm_v, blk_a, blk_b, blk_c, blk_d, stage_v,
             sem_a, sem_b, sem_c, sem_d):
    wid = lax.axis_index("s") * _NC + lax.axis_index("c")
    rowbase = wid * _ROWS_W
    pixbase = wid * _PIX_W

    pltpu.sync_copy(gt_hbm.at[pl.ds(rowbase, _ROWS_W)], gt_v)
    pltpu.sync_copy(vm_hbm.at[pl.ds(rowbase, _ROWS_W)], vm_v)

    lane = lax.iota(jnp.int32, _L)
    bufs = (blk_a, blk_b, blk_c, blk_d)
    sems = (sem_a, sem_b, sem_c, sem_d)

    def _copy(j, b):
        return pltpu.make_async_copy(
            pred_hbm.at[pl.ds(pixbase + j * _CHUNK, _CHUNK)],
            bufs[b], sems[b])

    for b in range(_RING):
        _copy(b, b).start()

    zero = jnp.zeros((_L,), jnp.float32)

    def _round(p, carry):
        ls, vs = carry
        for b in range(_RING):
            j = _RING * p + b
            _copy(j, b).wait()

            def _row(ri, carry):
                ls, vs = carry
                g_row = j * _CHUNK_ROWS + ri
                for gi, w0 in enumerate(_G0):
                    p_loc = ri * _W + w0 + lane
                    g = gt_v[g_row, pl.ds(w0, _L)]
                    bin_ = jnp.minimum(jnp.maximum(2 * g - 2, 0), _C - 1)
                    x = plsc.load_gather(bufs[b], [p_loc, bin_]) + 1e-8
                    vm = vm_v[g_row, pl.ds(w0, _L)]
                    if gi == len(_G0) - 1:
                        vm = jnp.where(lane >= 8, vm, 0.0)
                    ls = ls + vm * _log_f32(x)
                    vs = vs + vm
                return ls, vs

            ls, vs = lax.fori_loop(0, _CHUNK_ROWS, _row, (ls, vs))

            @pl.when(j + _RING < _NCHUNK)
            def _():
                _copy(j + _RING, b).start()
        return ls, vs

    ls, vs = lax.fori_loop(0, _NCHUNK // _RING, _round, (zero, zero))

    stage_v[pl.ds(0, _L)] = ls
    pltpu.sync_copy(stage_v, out_hbm.at[0, wid])
    stage_v[pl.ds(0, _L)] = vs
    pltpu.sync_copy(stage_v, out_hbm.at[1, wid])


_sc_call = pl.kernel(
    _sc_body,
    out_type=jax.ShapeDtypeStruct((2, _NW, _L), jnp.float32),
    mesh=plsc.VectorSubcoreMesh(
        core_axis_name="c", subcore_axis_name="s",
        num_cores=_NC, num_subcores=_NS),
    compiler_params=pltpu.CompilerParams(needs_layout_passes=False),
    scratch_types=[
        pltpu.VMEM((_ROWS_W, _W), jnp.int32),
        pltpu.VMEM((_ROWS_W, _W), jnp.float32),
        pltpu.VMEM((_CHUNK, _C), jnp.float32),
        pltpu.VMEM((_CHUNK, _C), jnp.float32),
        pltpu.VMEM((_CHUNK, _C), jnp.float32),
        pltpu.VMEM((_CHUNK, _C), jnp.float32),
        pltpu.VMEM((_L,), jnp.float32),
        pltpu.SemaphoreType.DMA,
        pltpu.SemaphoreType.DMA,
        pltpu.SemaphoreType.DMA,
        pltpu.SemaphoreType.DMA,
    ],
)


def _tc_body(pred_ref, gt_ref, vm_ref, out_ref):
    p3 = pred_ref[...].reshape(_TC_ROWS, _W, _C)
    g = gt_ref[...]
    vm = vm_ref[...]
    bins = jnp.minimum(jnp.maximum(2 * g - 2, 0), _C - 1)
    chan = lax.broadcasted_iota(jnp.int32, (_TC_ROWS, _W, _C), 2)
    v = jnp.sum(jnp.where(chan == bins[:, :, None], p3, 0.0), axis=2) + 1e-8
    ls = jnp.sum(vm * jnp.log(v))
    vs = jnp.sum(vm)
    lanes = lax.broadcasted_iota(jnp.int32, (1, 128), 1)
    row = jnp.where(lanes == 0, ls, jnp.where(lanes == 1, vs, 0.0))
    out_ref[pl.ds(pl.program_id(0), 1), :] = row


_tc_call = pl.pallas_call(
    _tc_body,
    grid=(_TC_GRID,),
    in_specs=[
        pl.BlockSpec((_TC_BLK, _C), lambda i: (_SC_PIX // _TC_BLK + i, 0)),
        pl.BlockSpec((_TC_ROWS, _W), lambda i: (_SC_ROWBLK + i, 0)),
        pl.BlockSpec((_TC_ROWS, _W), lambda i: (_SC_ROWBLK + i, 0)),
    ],
    out_specs=pl.BlockSpec((_TC_GRID, 128), lambda i: (0, 0)),
    out_shape=jax.ShapeDtypeStruct((_TC_GRID, 128), jnp.float32),
)


@jax.jit
def kernel(pred_depth, gt_depth_map, valid_mask):
    table = pred_depth.transpose(0, 2, 3, 1).reshape(_NPIX, _C)
    gt2 = gt_depth_map.reshape(_NROW, _W).astype(jnp.int32)
    vm2 = valid_mask.reshape(_NROW, _W)
    sc_parts = _sc_call(table, gt2, vm2)
    tc_parts = _tc_call(table, gt2, vm2)
    neg_wsum = jnp.sum(sc_parts[0]) + jnp.sum(tc_parts[:, 0])
    vm_sum = jnp.sum(sc_parts[1]) + jnp.sum(tc_parts[:, 1])
    weighted = -neg_wsum / jnp.maximum(vm_sum, 1e-12)
    return jnp.where(vm_sum > 0, weighted, jnp.float32(0.0))
